# baseline (speedup 1.0000x reference)
import jax
import jax.numpy as jnp
from jax.experimental import pallas as pl
from jax.experimental.pallas import tpu as pltpu

_X_CHUNKS = 4
_EA_CHUNKS = 16


def _dma_copy_body(x_hbm, ea_hbm, x_out, ea_out, x_sem, ea_sem):
    x_rows = 10000 // _X_CHUNKS
    ea_rows = 40000 // _EA_CHUNKS
    copies = []
    for i in range(_X_CHUNKS):
        copies.append(pltpu.make_async_copy(
            x_hbm.at[pl.ds(i * x_rows, x_rows), :],
            x_out.at[pl.ds(i * x_rows, x_rows), :],
            x_sem))
    for i in range(_EA_CHUNKS):
        copies.append(pltpu.make_async_copy(
            ea_hbm.at[pl.ds(i * ea_rows, ea_rows), :],
            ea_out.at[pl.ds(i * ea_rows, ea_rows), :],
            ea_sem))
    for c in copies:
        c.start()
    for c in copies:
        c.wait()


def kernel(x, edge_index, edge_attr):
    del edge_index
    ea2 = edge_attr.reshape(40000, 128)
    x_out, ea_out = pl.pallas_call(
        _dma_copy_body,
        in_specs=[
            pl.BlockSpec(memory_space=pl.ANY),
            pl.BlockSpec(memory_space=pl.ANY),
        ],
        out_specs=[
            pl.BlockSpec(memory_space=pl.ANY),
            pl.BlockSpec(memory_space=pl.ANY),
        ],
        out_shape=[
            jax.ShapeDtypeStruct((10000, 128), jnp.float32),
            jax.ShapeDtypeStruct((40000, 128), jnp.float32),
        ],
        scratch_shapes=[pltpu.SemaphoreType.DMA, pltpu.SemaphoreType.DMA],
    )(x, ea2)
    return (x_out, ea_out.reshape(320000, 16))


# speedup vs baseline: 3.4972x; 3.4972x over previous
import jax
import jax.numpy as jnp
from jax.experimental import pallas as pl
from jax.experimental.pallas import tpu as pltpu

_ROWS_X = 10000
_ROWS_EA = 40000
_CHUNK = 2500
_NX = _ROWS_X // _CHUNK
_NEA = _ROWS_EA // _CHUNK
_N = _NX + _NEA


def _copy_body(x_hbm, ea_hbm, x_out, ea_out,
               x_vmem, ea_vmem, load_sems, store_sems):
    loads = []
    stores = []
    for i in range(_NX):
        sl = pl.ds(i * _CHUNK, _CHUNK)
        loads.append(pltpu.make_async_copy(
            x_hbm.at[sl, :], x_vmem.at[sl, :], load_sems.at[i]))
        stores.append(pltpu.make_async_copy(
            x_vmem.at[sl, :], x_out.at[sl, :], store_sems.at[i]))
    for i in range(_NEA):
        sl = pl.ds(i * _CHUNK, _CHUNK)
        loads.append(pltpu.make_async_copy(
            ea_hbm.at[sl, :], ea_vmem.at[sl, :], load_sems.at[_NX + i]))
        stores.append(pltpu.make_async_copy(
            ea_vmem.at[sl, :], ea_out.at[sl, :], store_sems.at[_NX + i]))
    for ld in loads:
        ld.start()
    for ld, st in zip(loads, stores):
        ld.wait()
        st.start()
    for st in stores:
        st.wait()


def kernel(x, edge_index, edge_attr):
    del edge_index
    ea2 = edge_attr.reshape(_ROWS_EA, 128)
    x_out, ea_out = pl.pallas_call(
        _copy_body,
        in_specs=[
            pl.BlockSpec(memory_space=pl.ANY),
            pl.BlockSpec(memory_space=pl.ANY),
        ],
        out_specs=[
            pl.BlockSpec(memory_space=pl.ANY),
            pl.BlockSpec(memory_space=pl.ANY),
        ],
        out_shape=[
            jax.ShapeDtypeStruct((_ROWS_X, 128), jnp.float32),
            jax.ShapeDtypeStruct((_ROWS_EA, 128), jnp.float32),
        ],
        scratch_shapes=[
            pltpu.VMEM((_ROWS_X, 128), jnp.float32),
            pltpu.VMEM((_ROWS_EA, 128), jnp.float32),
            pltpu.SemaphoreType.DMA((_N,)),
            pltpu.SemaphoreType.DMA((_N,)),
        ],
    )(x, ea2)
    return (x_out, ea_out.reshape(320000, 16))


# speedup vs baseline: 3.8206x; 1.0925x over previous
import jax
import jax.numpy as jnp
from jax.experimental import pallas as pl
from jax.experimental.pallas import tpu as pltpu

_ROWS_X = 10000
_ROWS_EA = 320000
_X_CHUNK = 2500
_EA_CHUNK = 5000
_NX = _ROWS_X // _X_CHUNK
_NEA = _ROWS_EA // _EA_CHUNK
_N = _NX + _NEA
_EA_BUFS = 16
_LAG = 8


def _copy_body(x_hbm, ea_hbm, x_out, ea_out,
               x_vmem, ea_vmem, load_sems, store_sems):
    loads = []
    stores = []
    for i in range(_NX):
        sl = pl.ds(i * _X_CHUNK, _X_CHUNK)
        loads.append(pltpu.make_async_copy(
            x_hbm.at[sl, :], x_vmem.at[sl, :], load_sems.at[i]))
        stores.append(pltpu.make_async_copy(
            x_vmem.at[sl, :], x_out.at[sl, :], store_sems.at[i]))
    for i in range(_NEA):
        sl = pl.ds(i * _EA_CHUNK, _EA_CHUNK)
        buf = pl.ds((i % _EA_BUFS) * _EA_CHUNK, _EA_CHUNK)
        k = _NX + i
        loads.append(pltpu.make_async_copy(
            ea_hbm.at[sl, :], ea_vmem.at[buf, :], load_sems.at[k]))
        stores.append(pltpu.make_async_copy(
            ea_vmem.at[buf, :], ea_out.at[sl, :], store_sems.at[k]))

    store_waited = [False] * _N
    for i in range(_N + _LAG):
        if i < _N:
            reuse = i - _EA_BUFS
            if reuse >= _NX:
                stores[reuse].wait()
                store_waited[reuse] = True
            loads[i].start()
        j = i - _LAG
        if 0 <= j < _N:
            loads[j].wait()
            stores[j].start()
    for i in range(_N):
        if not store_waited[i]:
            stores[i].wait()


def kernel(x, edge_index, edge_attr):
    del edge_index
    x_out, ea_out = pl.pallas_call(
        _copy_body,
        in_specs=[
            pl.BlockSpec(memory_space=pl.ANY),
            pl.BlockSpec(memory_space=pl.ANY),
        ],
        out_specs=[
            pl.BlockSpec(memory_space=pl.ANY),
            pl.BlockSpec(memory_space=pl.ANY),
        ],
        out_shape=[
            jax.ShapeDtypeStruct((_ROWS_X, 128), jnp.float32),
            jax.ShapeDtypeStruct((_ROWS_EA, 16), jnp.float32),
        ],
        scratch_shapes=[
            pltpu.VMEM((_ROWS_X, 128), jnp.float32),
            pltpu.VMEM((_EA_BUFS * _EA_CHUNK, 16), jnp.float32),
            pltpu.SemaphoreType.DMA((_N,)),
            pltpu.SemaphoreType.DMA((_N,)),
        ],
    )(x, edge_attr)
    return (x_out, ea_out)


# speedup vs baseline: 3.8504x; 1.0078x over previous
import jax
import jax.numpy as jnp
from jax.experimental import pallas as pl
from jax.experimental.pallas import tpu as pltpu

_ROWS_X = 10000
_ROWS_EA = 320000
_X_CHUNK = 2500
_EA_CHUNK = 5000
_NX = _ROWS_X // _X_CHUNK
_NEA = _ROWS_EA // _EA_CHUNK
_N = _NX + _NEA
_EA_BUFS = 16
_LAG = 8


def _copy_body(x_hbm, ea_hbm, x_out, ea_out,
               x_vmem, ea_vmem, load_sems, store_sems):
    loads = []
    stores = []
    for i in range(_NX):
        sl = pl.ds(i * _X_CHUNK, _X_CHUNK)
        loads.append(pltpu.make_async_copy(
            x_hbm.at[sl, :], x_vmem.at[sl, :], load_sems.at[i]))
        stores.append(pltpu.make_async_copy(
            x_vmem.at[sl, :], x_out.at[sl, :], store_sems.at[i]))
    for i in range(_NEA):
        sl = pl.ds(i * _EA_CHUNK, _EA_CHUNK)
        buf = pl.ds((i % _EA_BUFS) * _EA_CHUNK, _EA_CHUNK)
        k = _NX + i
        loads.append(pltpu.make_async_copy(
            ea_hbm.at[sl, :], ea_vmem.at[buf, :], load_sems.at[k]))
        stores.append(pltpu.make_async_copy(
            ea_vmem.at[buf, :], ea_out.at[sl, :], store_sems.at[k]))

    store_waited = [False] * _N
    for i in range(_N + _LAG):
        if i < _N:
            reuse = i - _EA_BUFS
            if reuse >= _NX:
                stores[reuse].wait()
                store_waited[reuse] = True
            loads[i].start(priority=i % 2)
        j = i - _LAG
        if 0 <= j < _N:
            loads[j].wait()
            stores[j].start(priority=j % 2)
    for i in range(_N):
        if not store_waited[i]:
            stores[i].wait()


def kernel(x, edge_index, edge_attr):
    del edge_index
    x_out, ea_out = pl.pallas_call(
        _copy_body,
        in_specs=[
            pl.BlockSpec(memory_space=pl.ANY),
            pl.BlockSpec(memory_space=pl.ANY),
        ],
        out_specs=[
            pl.BlockSpec(memory_space=pl.ANY),
            pl.BlockSpec(memory_space=pl.ANY),
        ],
        out_shape=[
            jax.ShapeDtypeStruct((_ROWS_X, 128), jnp.float32),
            jax.ShapeDtypeStruct((_ROWS_EA, 16), jnp.float32),
        ],
        scratch_shapes=[
            pltpu.VMEM((_ROWS_X, 128), jnp.float32),
            pltpu.VMEM((_EA_BUFS * _EA_CHUNK, 16), jnp.float32),
            pltpu.SemaphoreType.DMA((_N,)),
            pltpu.SemaphoreType.DMA((_N,)),
        ],
    )(x, edge_attr)
    return (x_out, ea_out)
